# baseline (speedup 1.0000x reference)
import functools

import jax
import jax.numpy as jnp
from jax import lax
from jax.experimental import pallas as pl
from jax.experimental.pallas import tpu as pltpu
from jax.experimental.pallas import tpu_sc as plsc

TOPK = 32
NCH = 256
CS = 128


def _to_u32(h):
    bi = lax.bitcast_convert_type(h, jnp.int32)
    bu = lax.bitcast_convert_type(h, jnp.uint32)
    return jnp.where(bi < 0, ~bu, bu | jnp.uint32(0x80000000))


def _from_u32(u):
    back = jnp.where(u >= jnp.uint32(0x80000000), u & jnp.uint32(0x7FFFFFFF), ~u)
    return lax.bitcast_convert_type(back, jnp.float32)


def _encode_body(x_ref, enc_ref, bias_ref, out_ref):
    out_ref[...] = (
        jnp.dot(
            x_ref[...].astype(jnp.bfloat16),
            enc_ref[...].astype(jnp.bfloat16),
            preferred_element_type=jnp.float32,
        )
        + bias_ref[...]
    )


def _chunkmax_body(h_ref, m_ref, l_ref):
    tb = h_ref.shape[0]
    h = h_ref[...]
    m = jnp.max(h.reshape(tb, NCH, CS), axis=2)
    m_ref[...] = m
    u = _to_u32(m)

    def step(i, cand):
        bit = jnp.uint32(1) << (jnp.uint32(31) - i)
        t = cand | bit
        cnt = jnp.sum((u >= t).astype(jnp.int32), axis=1, keepdims=True)
        return jnp.where(cnt >= TOPK, t, cand)

    cand = lax.fori_loop(0, 32, step, jnp.zeros((tb, 1), jnp.uint32), unroll=True)
    l_ref[...] = jnp.broadcast_to(_from_u32(cand), l_ref.shape)


def _decode_body(h_ref, thr_ref, dec_ref, dbias_ref, out_ref):
    k = pl.program_id(1)
    thr = thr_ref[:, 0:1]
    a = h_ref[...]
    a = jnp.where(a >= thr, a, 0.0)
    a = jnp.maximum(a, 0.0)

    @pl.when(k == 0)
    def _():
        out_ref[...] = jnp.broadcast_to(dbias_ref[...], out_ref.shape)

    out_ref[...] += jnp.dot(
        a.astype(jnp.bfloat16),
        dec_ref[...].astype(jnp.bfloat16),
        preferred_element_type=jnp.float32,
    )


def _make_sc_select(B):
    info = plsc.get_sparse_core_info()
    NC, NS = info.num_cores, info.num_subcores
    NW = NC * NS
    TPW = B // NW
    CAND = NCH * CS
    mesh = plsc.VectorSubcoreMesh(core_axis_name="c", subcore_axis_name="s")

    @functools.partial(
        pl.kernel,
        mesh=mesh,
        out_type=jax.ShapeDtypeStruct((B, 128), jnp.float32),
        scratch_types=[
            pltpu.VMEM((TPW, NCH), jnp.float32),
            pltpu.VMEM((TPW, 128), jnp.float32),
            pltpu.VMEM((TPW, 128), jnp.float32),
            pltpu.VMEM((NCH,), jnp.int32),
            pltpu.VMEM((NCH, CS), jnp.float32),
            pltpu.VMEM((CAND + 32,), jnp.int32),
            pltpu.SemaphoreType.DMA,
        ],
        compiler_params=pltpu.CompilerParams(needs_layout_passes=False),
    )
    def sc_select(h3_hbm, m_hbm, l_hbm, thr_hbm, m_v, l_v, thr_v, cid_v, ch_v, cand_v, sem):
        wid = lax.axis_index("s") * NC + lax.axis_index("c")
        base = wid * TPW
        lane = lax.iota(jnp.int32, 16)

        pltpu.sync_copy(m_hbm.at[pl.ds(base, TPW)], m_v)
        pltpu.sync_copy(l_hbm.at[pl.ds(base, TPW)], l_v)

        def token_body(t, carry):
            lvec = l_v[t, 0:16]
            grow = (base + t) * NCH

            def cid_step(j, off):
                mvals = m_v[t, pl.ds(j * 16, 16)]
                mask = mvals >= lvec
                ids = grow + j * 16 + lane
                pos = plsc.cumsum(jnp.where(mask, jnp.int32(1), jnp.int32(0))) - 1 + off
                plsc.store_scatter(cid_v, [pos], ids, mask=mask)
                return off + plsc.all_reduce_population_count(mask)[0]

            qcnt = lax.fori_loop(0, NCH // 16, cid_step, jnp.int32(0), unroll=True)

            def gather_batch(b):
                @pl.when(b * 16 < qcnt)
                def _():
                    cidv = cid_v[pl.ds(b * 16, 16)]
                    pltpu.async_copy(h3_hbm.at[cidv], ch_v.at[pl.ds(b * 16, 16)], sem)

            def wait_batch(b):
                @pl.when(b * 16 < qcnt)
                def _():
                    pltpu.make_async_copy(
                        h3_hbm.at[cid_v[pl.ds(b * 16, 16)]],
                        ch_v.at[pl.ds(b * 16, 16)],
                        sem,
                    ).wait()

            for b in range(NCH // 16):
                gather_batch(b)
            for b in range(NCH // 16):
                wait_batch(b)

            def cand_chunk(j, cnt):
                for v in range(CS // 16):
                    x = ch_v[j, pl.ds(v * 16, 16)]
                    mask = x >= lvec
                    u = _to_u32(x)
                    pos = plsc.cumsum(jnp.where(mask, jnp.int32(1), jnp.int32(0))) - 1 + cnt
                    plsc.store_scatter(cand_v, [pos], plsc.bitcast(u, jnp.int32), mask=mask)
                    cnt = cnt + plsc.all_reduce_population_count(mask)[0]
                return cnt

            ccnt = lax.fori_loop(0, qcnt, cand_chunk, jnp.int32(0))
            cand_v[pl.ds(ccnt, 16)] = jnp.zeros((16,), jnp.int32)
            nv = (ccnt + 15) // 16

            def bit_step(i, cand):
                bit = jnp.uint32(1) << (jnp.uint32(31) - i.astype(jnp.uint32))
                tt = cand | bit

                def cnt_step(j, acc):
                    uvec = plsc.bitcast(cand_v[pl.ds(j * 16, 16)], jnp.uint32)
                    return acc + jnp.where(uvec >= tt, jnp.int32(1), jnp.int32(0))

                accv = lax.fori_loop(0, nv, cnt_step, jnp.zeros((16,), jnp.int32))
                cnt = jnp.sum(accv)
                return jnp.where(cnt >= TOPK, tt, cand)

            v32u = lax.fori_loop(0, 32, bit_step, jnp.uint32(0))
            thrv = jnp.broadcast_to(_from_u32(v32u), (16,))
            for j in range(8):
                thr_v[t, pl.ds(j * 16, 16)] = thrv
            return carry

        lax.fori_loop(0, TPW, token_body, jnp.int32(0))
        pltpu.sync_copy(thr_v, thr_hbm.at[pl.ds(base, TPW)])

    return sc_select


def kernel(x, encoder, encoder_bias, decoder, decoder_bias):
    B, D = x.shape
    H = encoder.shape[1]

    M_BLK = min(1024, B)
    H_BLK = min(512, H)
    TB = min(64, B)

    h = pl.pallas_call(
        _encode_body,
        grid=(B // M_BLK, H // H_BLK),
        in_specs=[
            pl.BlockSpec((M_BLK, D), lambda m, hb: (m, 0)),
            pl.BlockSpec((D, H_BLK), lambda m, hb: (0, hb)),
            pl.BlockSpec((1, H_BLK), lambda m, hb: (0, hb)),
        ],
        out_specs=pl.BlockSpec((M_BLK, H_BLK), lambda m, hb: (m, hb)),
        out_shape=jax.ShapeDtypeStruct((B, H), jnp.float32),
    )(x, encoder, encoder_bias.reshape(1, H))

    m, l = pl.pallas_call(
        _chunkmax_body,
        grid=(B // TB,),
        in_specs=[pl.BlockSpec((TB, H), lambda tb: (tb, 0))],
        out_specs=[
            pl.BlockSpec((TB, NCH), lambda tb: (tb, 0)),
            pl.BlockSpec((TB, 128), lambda tb: (tb, 0)),
        ],
        out_shape=[
            jax.ShapeDtypeStruct((B, NCH), jnp.float32),
            jax.ShapeDtypeStruct((B, 128), jnp.float32),
        ],
    )(h)

    h3 = h.reshape(B * NCH, CS)
    thr = _make_sc_select(B)(h3, m, l)

    recon = pl.pallas_call(
        _decode_body,
        grid=(B // M_BLK, H // H_BLK),
        in_specs=[
            pl.BlockSpec((M_BLK, H_BLK), lambda mm, k: (mm, k)),
            pl.BlockSpec((M_BLK, 128), lambda mm, k: (mm, 0)),
            pl.BlockSpec((H_BLK, D), lambda mm, k: (k, 0)),
            pl.BlockSpec((1, D), lambda mm, k: (0, 0)),
        ],
        out_specs=pl.BlockSpec((M_BLK, D), lambda mm, k: (mm, 0)),
        out_shape=jax.ShapeDtypeStruct((B, D), jnp.float32),
        compiler_params=pltpu.CompilerParams(
            dimension_semantics=("parallel", "arbitrary")
        ),
    )(h, thr, decoder, decoder_bias.reshape(1, D))

    return recon


# speedup vs baseline: 2.1947x; 2.1947x over previous
import functools

import jax
import jax.numpy as jnp
from jax import lax
from jax.experimental import pallas as pl
from jax.experimental.pallas import tpu as pltpu
from jax.experimental.pallas import tpu_sc as plsc

TOPK = 32
NCH = 256
CS = 128


def _to_u32(h):
    bi = lax.bitcast_convert_type(h, jnp.int32)
    bu = lax.bitcast_convert_type(h, jnp.uint32)
    return jnp.where(bi < 0, ~bu, bu | jnp.uint32(0x80000000))


def _from_u32(u):
    back = jnp.where(u >= jnp.uint32(0x80000000), u & jnp.uint32(0x7FFFFFFF), ~u)
    return lax.bitcast_convert_type(back, jnp.float32)


def _encode_body(x_ref, enc_ref, bias_ref, out_ref):
    out_ref[...] = (
        jnp.dot(
            x_ref[...].astype(jnp.bfloat16),
            enc_ref[...].astype(jnp.bfloat16),
            preferred_element_type=jnp.float32,
        )
        + bias_ref[...]
    )


def _chunkmax_body(h_ref, m_ref):
    tb = h_ref.shape[0]
    h = h_ref[...]
    m_ref[...] = jnp.max(h.reshape(tb * NCH, CS), axis=1, keepdims=True)


def _decode_body(h_ref, thr_ref, dec_ref, dbias_ref, out_ref):
    k = pl.program_id(1)
    thr = thr_ref[:, 0:1]
    a = h_ref[...]
    a = jnp.where(a >= thr, a, 0.0)
    a = jnp.maximum(a, 0.0)

    @pl.when(k == 0)
    def _():
        out_ref[...] = jnp.broadcast_to(dbias_ref[...], out_ref.shape)

    out_ref[...] += jnp.dot(
        a.astype(jnp.bfloat16),
        dec_ref[...].astype(jnp.bfloat16),
        preferred_element_type=jnp.float32,
    )


def _make_sc_select(B):
    info = plsc.get_sparse_core_info()
    NC, NS = info.num_cores, info.num_subcores
    NW = NC * NS
    TPW = B // NW
    CAND = NCH * CS
    mesh = plsc.VectorSubcoreMesh(core_axis_name="c", subcore_axis_name="s")

    @functools.partial(
        pl.kernel,
        mesh=mesh,
        out_type=jax.ShapeDtypeStruct((B, 128), jnp.float32),
        scratch_types=[
            pltpu.VMEM((TPW, NCH), jnp.float32),
            pltpu.VMEM((TPW, 128), jnp.float32),
            pltpu.VMEM((NCH,), jnp.int32),
            pltpu.VMEM((NCH, CS), jnp.float32),
            pltpu.VMEM((CAND + 32,), jnp.int32),
            pltpu.SemaphoreType.DMA,
        ],
        compiler_params=pltpu.CompilerParams(needs_layout_passes=False),
    )
    def sc_select(h3_hbm, m_hbm, thr_hbm, m_v, thr_v, cid_v, ch_v, cand_v, sem):
        wid = lax.axis_index("s") * NC + lax.axis_index("c")
        base = wid * TPW
        lane = lax.iota(jnp.int32, 16)

        pltpu.sync_copy(m_hbm.at[pl.ds(base, TPW)], m_v)

        def token_body(t, carry):
            grow = (base + t) * NCH
            mus = [_to_u32(m_v[t, pl.ds(j * 16, 16)]) for j in range(NCH // 16)]

            def l_bit_step(i, cand):
                bit = jnp.uint32(1) << (jnp.uint32(31) - i.astype(jnp.uint32))
                tt = cand | bit
                accv = jnp.zeros((16,), jnp.int32)
                for j in range(NCH // 16):
                    accv = accv + jnp.where(mus[j] >= tt, jnp.int32(1), jnp.int32(0))
                return jnp.where(jnp.sum(accv) >= TOPK, tt, cand)

            lu = lax.fori_loop(0, 32, l_bit_step, jnp.uint32(0))
            lvec = jnp.broadcast_to(_from_u32(lu), (16,))

            qcnt = jnp.int32(0)
            for j in range(NCH // 16):
                mask = mus[j] >= lu
                ids = grow + j * 16 + lane
                pos = plsc.cumsum(jnp.where(mask, jnp.int32(1), jnp.int32(0))) - 1 + qcnt
                plsc.store_scatter(cid_v, [pos], ids, mask=mask)
                qcnt = qcnt + plsc.all_reduce_population_count(mask)[0]

            def gather_batch(b):
                @pl.when(b * 16 < qcnt)
                def _():
                    cidv = cid_v[pl.ds(b * 16, 16)]
                    pltpu.async_copy(h3_hbm.at[cidv], ch_v.at[pl.ds(b * 16, 16)], sem)

            def wait_batch(b):
                @pl.when(b * 16 < qcnt)
                def _():
                    pltpu.make_async_copy(
                        h3_hbm.at[cid_v[pl.ds(b * 16, 16)]],
                        ch_v.at[pl.ds(b * 16, 16)],
                        sem,
                    ).wait()

            for b in range(NCH // 16):
                gather_batch(b)
            for b in range(NCH // 16):
                wait_batch(b)

            def cand_chunk(j, cnt):
                for v in range(CS // 16):
                    x = ch_v[j, pl.ds(v * 16, 16)]
                    mask = x >= lvec
                    u = _to_u32(x)
                    pos = plsc.cumsum(jnp.where(mask, jnp.int32(1), jnp.int32(0))) - 1 + cnt
                    plsc.store_scatter(cand_v, [pos], plsc.bitcast(u, jnp.int32), mask=mask)
                    cnt = cnt + plsc.all_reduce_population_count(mask)[0]
                return cnt

            ccnt = lax.fori_loop(0, qcnt, cand_chunk, jnp.int32(0))
            cand_v[pl.ds(ccnt, 16)] = jnp.zeros((16,), jnp.int32)
            nv = (ccnt + 15) // 16

            def bit_step(i, cand):
                bit = jnp.uint32(1) << (jnp.uint32(31) - i.astype(jnp.uint32))
                tt = cand | bit

                def cnt_step(j, acc):
                    uvec = plsc.bitcast(cand_v[pl.ds(j * 16, 16)], jnp.uint32)
                    return acc + jnp.where(uvec >= tt, jnp.int32(1), jnp.int32(0))

                accv = lax.fori_loop(0, nv, cnt_step, jnp.zeros((16,), jnp.int32))
                cnt = jnp.sum(accv)
                return jnp.where(cnt >= TOPK, tt, cand)

            v32u = lax.fori_loop(0, 32, bit_step, jnp.uint32(0))
            thrv = jnp.broadcast_to(_from_u32(v32u), (16,))
            for j in range(8):
                thr_v[t, pl.ds(j * 16, 16)] = thrv
            return carry

        lax.fori_loop(0, TPW, token_body, jnp.int32(0))
        pltpu.sync_copy(thr_v, thr_hbm.at[pl.ds(base, TPW)])

    return sc_select


def kernel(x, encoder, encoder_bias, decoder, decoder_bias):
    B, D = x.shape
    H = encoder.shape[1]

    M_BLK = min(1024, B)
    H_BLK = min(512, H)
    TB = min(64, B)

    h = pl.pallas_call(
        _encode_body,
        grid=(B // M_BLK, H // H_BLK),
        in_specs=[
            pl.BlockSpec((M_BLK, D), lambda m, hb: (m, 0)),
            pl.BlockSpec((D, H_BLK), lambda m, hb: (0, hb)),
            pl.BlockSpec((1, H_BLK), lambda m, hb: (0, hb)),
        ],
        out_specs=pl.BlockSpec((M_BLK, H_BLK), lambda m, hb: (m, hb)),
        out_shape=jax.ShapeDtypeStruct((B, H), jnp.float32),
    )(x, encoder, encoder_bias.reshape(1, H))

    m = pl.pallas_call(
        _chunkmax_body,
        grid=(B // TB,),
        in_specs=[pl.BlockSpec((TB, H), lambda tb: (tb, 0))],
        out_specs=pl.BlockSpec((TB * NCH, 1), lambda tb: (tb, 0)),
        out_shape=jax.ShapeDtypeStruct((B * NCH, 1), jnp.float32),
    )(h)

    h3 = h.reshape(B * NCH, CS)
    thr = _make_sc_select(B)(h3, m.reshape(B, NCH))

    recon = pl.pallas_call(
        _decode_body,
        grid=(B // M_BLK, H // H_BLK),
        in_specs=[
            pl.BlockSpec((M_BLK, H_BLK), lambda mm, k: (mm, k)),
            pl.BlockSpec((M_BLK, 128), lambda mm, k: (mm, 0)),
            pl.BlockSpec((H_BLK, D), lambda mm, k: (k, 0)),
            pl.BlockSpec((1, D), lambda mm, k: (0, 0)),
        ],
        out_specs=pl.BlockSpec((M_BLK, D), lambda mm, k: (mm, 0)),
        out_shape=jax.ShapeDtypeStruct((B, D), jnp.float32),
        compiler_params=pltpu.CompilerParams(
            dimension_semantics=("parallel", "arbitrary")
        ),
    )(h, thr, decoder, decoder_bias.reshape(1, D))

    return recon
